# baseline (speedup 1.0000x reference)
import functools
import math

import jax
import jax.numpy as jnp
from jax import lax
from jax.experimental import pallas as pl
from jax.experimental.pallas import tpu as pltpu
from jax.experimental.pallas import tpu_sc as plsc

D = 256
SCALE = math.sqrt(D)
NC = 2
NS = 16
NW = NC * NS
N = 1024 * 200
TPW = N // NW
C = 64
NCH = TPW // C
LANES = 16
V_ROWS = 259
C_ROWS = 4

_mesh = plsc.VectorSubcoreMesh(core_axis_name="c", subcore_axis_name="s")


def _pack_pairs(table):
    rows = table.shape[0]
    t = table.reshape(rows, D // 32, 2, LANES).transpose(0, 1, 3, 2)
    bf = t.astype(jnp.bfloat16)
    return jax.lax.bitcast_convert_type(bf, jnp.int32).reshape(rows, D // 2)


@functools.partial(
    pl.kernel,
    out_type=jax.ShapeDtypeStruct((NW, NCH, C, D), jnp.float32),
    mesh=_mesh,
    scratch_types=[
        pltpu.VMEM((NCH, C), jnp.int32),
        pltpu.VMEM((NCH, C), jnp.int32),
        pltpu.VMEM((NCH, C), jnp.int32),
        pltpu.VMEM((V_ROWS, D // 2), jnp.int32),
        pltpu.VMEM((C_ROWS, D // 2), jnp.int32),
        pltpu.VMEM((C, D), jnp.float32),
        pltpu.VMEM((C, D), jnp.float32),
        pltpu.SMEM((LANES,), jnp.int32),
        pltpu.SMEM((LANES,), jnp.int32),
        pltpu.SemaphoreType.DMA,
        pltpu.SemaphoreType.DMA,
        pltpu.SemaphoreType.DMA,
        pltpu.SemaphoreType.DMA,
    ],
)
def _embed_kernel(v_hbm, c_hbm, p_hbm, vt, ct, pt, out,
                  vi, ci, pi, vt_l, ct_l, pbuf, obuf, vsm, csm,
                  gsem0, gsem1, osem0, osem1):
    HC = C // 2
    H = NCH * 2
    gsems = (gsem0, gsem1)
    osems = (osem0, osem1)
    wid = lax.axis_index("s") * NC + lax.axis_index("c")
    pltpu.sync_copy(v_hbm.at[wid], vi)
    pltpu.sync_copy(c_hbm.at[wid], ci)
    pltpu.sync_copy(p_hbm.at[wid], pi)
    pltpu.sync_copy(vt, vt_l)
    pltpu.sync_copy(ct, ct_l)

    def _both_halves(w):
        lo = jax.lax.bitcast_convert_type(w << 16, jnp.float32)
        hi = jax.lax.bitcast_convert_type(w, jnp.float32)
        return lo, hi

    pltpu.async_copy(pt.at[pi.at[0, pl.ds(0, HC)]],
                     pbuf.at[pl.ds(0, HC)], gsems[0])

    def chunk_body(j, carry):
        for p in (0, 1):
            h = 2 * j + p
            q = 1 - p
            jn = j + p
            off = HC if p == 0 else 0

            @pl.when(h + 1 < H)
            def _fire_next():
                pltpu.async_copy(pt.at[pi.at[jn, pl.ds(off, HC)]],
                                 pbuf.at[pl.ds(q * HC, HC)], gsems[q])

            pltpu.make_async_copy(pt.at[pl.ds(0, HC)],
                                  pbuf.at[pl.ds(p * HC, HC)],
                                  gsems[p]).wait()
            @pl.when(h >= 2)
            def _drain_out():
                pltpu.make_async_copy(obuf.at[pl.ds(p * HC, HC)],
                                      out.at[wid, j, pl.ds(p * HC, HC)],
                                      osems[p]).wait()

            for g in range(HC // LANES):
                t0 = p * HC + g * LANES
                vvec = vi[j, pl.ds(t0, LANES)]
                cvec = ci[j, pl.ds(t0, LANES)]
                for l in range(LANES):
                    vsm[l] = vvec[l]
                    csm[l] = cvec[l]

                @plsc.parallel_loop(0, LANES, unroll=4)
                def tok_body(k):
                    t = t0 + k
                    v = vsm[k]
                    c = csm[k]
                    for u2 in range(D // 32):
                        sw = pl.ds(u2 * LANES, LANES)
                        va, vb = _both_halves(vt_l[v, sw])
                        ca, cb = _both_halves(ct_l[c, sw])
                        slo = pl.ds(32 * u2, LANES)
                        shi = pl.ds(32 * u2 + LANES, LANES)
                        obuf[t, slo] = (pbuf[t, slo] + va + ca) * SCALE
                        obuf[t, shi] = (pbuf[t, shi] + vb + cb) * SCALE

            pltpu.async_copy(obuf.at[pl.ds(p * HC, HC)],
                             out.at[wid, j, pl.ds(p * HC, HC)], osems[p])
        return carry

    lax.fori_loop(0, NCH, chunk_body, 0, unroll=False)
    for p in (0, 1):
        pltpu.make_async_copy(obuf.at[pl.ds(p * HC, HC)],
                              out.at[wid, NCH - 1, pl.ds(p * HC, HC)],
                              osems[p]).wait()


def kernel(value_tokens, coord_type_tokens, position_tokens,
           value_table, coord_table, pos_table):
    v3 = value_tokens.reshape(NW, NCH, C).astype(jnp.int32)
    c3 = coord_type_tokens.reshape(NW, NCH, C).astype(jnp.int32)
    p3 = position_tokens.reshape(NW, NCH, C).astype(jnp.int32)
    out = _embed_kernel(v3, c3, p3,
                        _pack_pairs(value_table),
                        _pack_pairs(coord_table),
                        pos_table)
    return out.reshape(value_tokens.shape[0], value_tokens.shape[1], D)


# speedup vs baseline: 1.0826x; 1.0826x over previous
import functools
import math

import jax
import jax.numpy as jnp
from jax import lax
from jax.experimental import pallas as pl
from jax.experimental.pallas import tpu as pltpu
from jax.experimental.pallas import tpu_sc as plsc

D = 256
SCALE = math.sqrt(D)
NC = 2
NS = 16
NW = NC * NS
N = 1024 * 200
TPW = N // NW
C = 64
NCH = TPW // C
LANES = 16
V_ROWS = 259
C_ROWS = 4

_mesh = plsc.VectorSubcoreMesh(core_axis_name="c", subcore_axis_name="s")


def _pack_pairs(table):
    rows = table.shape[0]
    t = table.reshape(rows, D // 32, 2, LANES).transpose(0, 1, 3, 2)
    bf = t.astype(jnp.bfloat16)
    return jax.lax.bitcast_convert_type(bf, jnp.int32).reshape(rows, D // 2)


@functools.partial(
    pl.kernel,
    out_type=jax.ShapeDtypeStruct((NW, NCH, C, D), jnp.float32),
    mesh=_mesh,
    scratch_types=[
        pltpu.VMEM((NCH, C), jnp.int32),
        pltpu.VMEM((NCH, C), jnp.int32),
        pltpu.VMEM((NCH, C), jnp.int32),
        pltpu.VMEM((V_ROWS, D // 2), jnp.int32),
        pltpu.VMEM((C_ROWS, D // 2), jnp.int32),
        pltpu.VMEM((C, D), jnp.float32),
        pltpu.VMEM((C, D), jnp.float32),
        pltpu.SMEM((LANES,), jnp.int32),
        pltpu.SMEM((LANES,), jnp.int32),
        pltpu.SemaphoreType.DMA,
        pltpu.SemaphoreType.DMA,
        pltpu.SemaphoreType.DMA,
        pltpu.SemaphoreType.DMA,
    ],
)
def _embed_kernel(v_hbm, c_hbm, p_hbm, vt, ct, pt, out,
                  vi, ci, pi, vt_l, ct_l, pbuf, obuf, vsm, csm,
                  gsem0, gsem1, osem0, osem1):
    HC = C // 2
    H = NCH * 2
    gsems = (gsem0, gsem1)
    osems = (osem0, osem1)
    wid = lax.axis_index("s") * NC + lax.axis_index("c")
    pltpu.sync_copy(v_hbm.at[wid], vi)
    pltpu.sync_copy(c_hbm.at[wid], ci)
    pltpu.sync_copy(p_hbm.at[wid], pi)
    pltpu.sync_copy(vt, vt_l)
    pltpu.sync_copy(ct, ct_l)

    def _both_halves(w):
        lo = jax.lax.bitcast_convert_type(w << 16, jnp.float32)
        hi = jax.lax.bitcast_convert_type(w & jnp.int32(-65536), jnp.float32)
        return lo, hi

    pltpu.async_copy(pt.at[pi.at[0, pl.ds(0, HC)]],
                     pbuf.at[pl.ds(0, HC)], gsems[0])

    def chunk_body(j, carry):
        for p in (0, 1):
            h = 2 * j + p
            q = 1 - p
            jn = j + p
            off = HC if p == 0 else 0

            @pl.when(h + 1 < H)
            def _fire_next():
                pltpu.async_copy(pt.at[pi.at[jn, pl.ds(off, HC)]],
                                 pbuf.at[pl.ds(q * HC, HC)], gsems[q])

            pltpu.make_async_copy(pt.at[pl.ds(0, HC)],
                                  pbuf.at[pl.ds(p * HC, HC)],
                                  gsems[p]).wait()
            @pl.when(h >= 2)
            def _drain_out():
                pltpu.make_async_copy(obuf.at[pl.ds(p * HC, HC)],
                                      out.at[wid, j, pl.ds(p * HC, HC)],
                                      osems[p]).wait()

            for g in range(HC // LANES):
                t0 = p * HC + g * LANES
                vvec = vi[j, pl.ds(t0, LANES)]
                cvec = ci[j, pl.ds(t0, LANES)]
                for l in range(LANES):
                    vsm[l] = vvec[l]
                    csm[l] = cvec[l]

                @plsc.parallel_loop(0, LANES, unroll=2)
                def tok_body(k):
                    t = t0 + k
                    v = vsm[k]
                    c = csm[k]
                    for u2 in range(D // 32):
                        sw = pl.ds(u2 * LANES, LANES)
                        va, vb = _both_halves(vt_l[v, sw])
                        ca, cb = _both_halves(ct_l[c, sw])
                        slo = pl.ds(32 * u2, LANES)
                        shi = pl.ds(32 * u2 + LANES, LANES)
                        obuf[t, slo] = (pbuf[t, slo] + va + ca) * SCALE
                        obuf[t, shi] = (pbuf[t, shi] + vb + cb) * SCALE

            pltpu.async_copy(obuf.at[pl.ds(p * HC, HC)],
                             out.at[wid, j, pl.ds(p * HC, HC)], osems[p])
        return carry

    lax.fori_loop(0, NCH, chunk_body, 0, unroll=False)
    for p in (0, 1):
        pltpu.make_async_copy(obuf.at[pl.ds(p * HC, HC)],
                              out.at[wid, NCH - 1, pl.ds(p * HC, HC)],
                              osems[p]).wait()


def kernel(value_tokens, coord_type_tokens, position_tokens,
           value_table, coord_table, pos_table):
    v3 = value_tokens.reshape(NW, NCH, C).astype(jnp.int32)
    c3 = coord_type_tokens.reshape(NW, NCH, C).astype(jnp.int32)
    p3 = position_tokens.reshape(NW, NCH, C).astype(jnp.int32)
    out = _embed_kernel(v3, c3, p3,
                        _pack_pairs(value_table),
                        _pack_pairs(coord_table),
                        pos_table)
    return out.reshape(value_tokens.shape[0], value_tokens.shape[1], D)
